# baseline (speedup 1.0000x reference)
import functools

import jax
import jax.numpy as jnp
from jax import lax
from jax.experimental import pallas as pl
from jax.experimental.pallas import tpu as pltpu
from jax.experimental.pallas import tpu_sc as plsc

N = 10000
E = 320000
D = 128
H = 16
NPAD = 10112
EPC = 128
CPT = 80
CF = 104
CS = 56
NTILES = 32
EPT = EPC * CPT
EPAD = EPT * NTILES
RPT = NPAD // 16


NB = 4


def _segsum_body(t_hbm, src_hbm, dst_hbm, out_hbm,
                 src_v, dst_v, rows_v, zbuf, acc_sh, t_sh,
                 gsem, gsem2, ssem, ssem2):
    c = lax.axis_index("c")
    s = lax.axis_index("s")

    pltpu.sync_copy(t_hbm.at[pl.ds(s * RPT, RPT)], t_sh.at[pl.ds(s * RPT, RPT)])

    def zrow(j, carry):
        zbuf[j, :] = jnp.zeros((16,), jnp.float32)
        return carry
    lax.fori_loop(0, RPT, zrow, 0)
    pltpu.sync_copy(zbuf, acc_sh.at[pl.ds(s * RPT, RPT)])

    @pl.when(c == 0)
    def _():
        pltpu.sync_copy(src_hbm.at[pl.ds(s * CF, CF)], src_v)
        pltpu.sync_copy(dst_hbm.at[pl.ds(s * CF, CF)], dst_v)

    @pl.when(c == 1)
    def _():
        pltpu.sync_copy(src_hbm.at[pl.ds(16 * CF + s * CS, CS)],
                        src_v.at[pl.ds(0, CS)])
        pltpu.sync_copy(dst_hbm.at[pl.ds(16 * CF + s * CS, CS)],
                        dst_v.at[pl.ds(0, CS)])
    plsc.subcore_barrier()

    ngrp2 = jnp.where(c == 0, CF // NB // 2, CS // NB // 2)

    def fire_gathers(grp, st, sem):
        for b in range(NB):
            pltpu.async_copy(t_sh.at[src_v.at[grp * NB + b]],
                             rows_v.at[st, b], sem)

    def drain(st, sem, kind):
        for b in range(NB):
            if kind == "g":
                pltpu.make_async_copy(t_sh.at[src_v.at[b]],
                                      rows_v.at[st, b], sem).wait()
            else:
                pltpu.make_async_copy(rows_v.at[st, b],
                                      acc_sh.at[dst_v.at[b]], sem).wait()

    def fire_scatters(grp, st, sem):
        for b in range(NB):
            pltpu.async_copy(rows_v.at[st, b],
                             acc_sh.at[dst_v.at[grp * NB + b]],
                             sem, add=True)

    fire_gathers(0, 0, gsem)

    def chunkgrp2(i, carry):
        g0 = 2 * i
        drain(0, gsem, "g")
        @pl.when(i >= 1)
        def _():
            drain(1, ssem2, "s")
        fire_gathers(g0 + 1, 1, gsem2)
        fire_scatters(g0, 0, ssem)
        drain(1, gsem2, "g")
        drain(0, ssem, "s")
        @pl.when(g0 + 2 < 2 * ngrp2)
        def _():
            fire_gathers(g0 + 2, 0, gsem)
        fire_scatters(g0 + 1, 1, ssem2)
        return carry
    lax.fori_loop(0, ngrp2, chunkgrp2, 0)
    drain(1, ssem2, "s")
    plsc.subcore_barrier()

    pltpu.sync_copy(acc_sh.at[pl.ds(s * RPT, RPT)], zbuf)
    pltpu.sync_copy(zbuf, out_hbm.at[c, pl.ds(s * RPT, RPT)])


_segsum = functools.partial(
    pl.kernel,
    _segsum_body,
    out_type=jax.ShapeDtypeStruct((2, NPAD, H), jnp.float32),
    mesh=plsc.VectorSubcoreMesh(core_axis_name="c", subcore_axis_name="s"),
    compiler_params=pltpu.CompilerParams(use_tc_tiling_on_sc=False),
    scratch_types=[
        pltpu.VMEM((CF, EPC), jnp.int32),
        pltpu.VMEM((CF, EPC), jnp.int32),
        pltpu.VMEM((2, NB, EPC, H), jnp.float32),
        pltpu.VMEM((RPT, H), jnp.float32),
        pltpu.VMEM_SHARED((NPAD, H), jnp.float32),
        pltpu.VMEM_SHARED((NPAD, H), jnp.float32),
        pltpu.SemaphoreType.DMA,
        pltpu.SemaphoreType.DMA,
        pltpu.SemaphoreType.DMA,
        pltpu.SemaphoreType.DMA,
    ],
)()


NW = NPAD // 8
NWVAL = N // 8


def _mm0_body(x8_ref, bdn_ref, bds_ref, t_ref, s_ref, xsum_ref):
    xx = x8_ref[...]
    t_ref[...] = jnp.dot(xx, bdn_ref[...], preferred_element_type=jnp.float32)
    s_ref[...] = jnp.dot(xx, bds_ref[...], preferred_element_type=jnp.float32)
    xsum_ref[...] = jnp.sum(xx, axis=0, keepdims=True)


def _layer_body(agg_ref, sprev_ref, bn_ref, bdn_ref, bds_ref,
                t_ref, s_ref, hsum_ref):
    h = jax.nn.relu(agg_ref[0] + agg_ref[1] + bn_ref[...] + sprev_ref[...])
    row = lax.broadcasted_iota(jnp.int32, (NW, 128), 0)
    hm = jnp.where(row < NWVAL, h, 0.0)
    hsum_ref[...] = jnp.sum(hm, axis=0, keepdims=True)
    t_ref[...] = jnp.dot(h, bdn_ref[...], preferred_element_type=jnp.float32)
    s_ref[...] = jnp.dot(h, bds_ref[...], preferred_element_type=jnp.float32)


def _final_body(agg_ref, sprev_ref, bn_ref, xsum_ref, h1s_ref, h2s_ref,
                wx_ref, w1_ref, w2_ref, w3_ref, fcb_ref, out_ref):
    h = jax.nn.relu(agg_ref[0] + agg_ref[1] + bn_ref[...] + sprev_ref[...])
    row = lax.broadcasted_iota(jnp.int32, (NW, 128), 0)
    hm = jnp.where(row < NWVAL, h, 0.0)
    h3sum = jnp.sum(hm, axis=0, keepdims=True)
    logits = (
        jnp.dot(xsum_ref[...], wx_ref[...], preferred_element_type=jnp.float32)
        + jnp.dot(h1s_ref[...], w1_ref[...], preferred_element_type=jnp.float32)
        + jnp.dot(h2s_ref[...], w2_ref[...], preferred_element_type=jnp.float32)
        + jnp.dot(h3sum, w3_ref[...], preferred_element_type=jnp.float32)
        + fcb_ref[...]
    )
    m = jnp.max(logits, axis=1, keepdims=True)
    lse = jnp.log(jnp.sum(jnp.exp(logits - m), axis=1, keepdims=True)) + m
    out_ref[...] = logits - lse


def _tc(body, out_shapes):
    return pl.pallas_call(body, out_shape=out_shapes)


def kernel(x, edge_index, Wn0, bn0, Ws0, Wn1, bn1, Ws1, Wn2, bn2, Ws2,
           fc_W, fc_b):
    f32 = jnp.float32
    x8 = jnp.pad(x, ((0, NPAD - N), (0, 0))).reshape(NW, 8 * D)
    pi = jnp.arange(EPAD - E, dtype=jnp.int32)
    src2d = jnp.concatenate([edge_index[0], pi % 16]).reshape(EPAD // EPC, EPC)
    dst2d = jnp.concatenate([edge_index[1], N + (pi % 16)]).reshape(EPAD // EPC, EPC)
    eye8 = jnp.eye(8, dtype=f32)
    bdn0, bds0 = jnp.kron(eye8, Wn0), jnp.kron(eye8, Ws0)
    bdn1, bds1 = jnp.kron(eye8, Wn1), jnp.kron(eye8, Ws1)
    bdn2, bds2 = jnp.kron(eye8, Wn2), jnp.kron(eye8, Ws2)
    bn0w = jnp.tile(bn0, 8).reshape(1, 128)
    bn1w = jnp.tile(bn1, 8).reshape(1, 128)
    bn2w = jnp.tile(bn2, 8).reshape(1, 128)
    fcb = fc_b.reshape(1, -1)
    wx = jnp.tile(fc_W[:D], (8, 1))
    w1 = jnp.tile(fc_W[D:D + H], (8, 1))
    w2 = jnp.tile(fc_W[D + H:D + 2 * H], (8, 1))
    w3 = jnp.tile(fc_W[D + 2 * H:], (8, 1))

    def narrow(a):
        return a.reshape(NPAD, H)

    def wide(a):
        return a.reshape(2, NW, 128)

    t0, s0, xsum = _tc(_mm0_body, (
        jax.ShapeDtypeStruct((NW, 128), f32),
        jax.ShapeDtypeStruct((NW, 128), f32),
        jax.ShapeDtypeStruct((1, 8 * D), f32),
    ))(x8, bdn0, bds0)

    agg0 = wide(_segsum(narrow(t0), src2d, dst2d))
    t1, s1, h1sum = _tc(_layer_body, (
        jax.ShapeDtypeStruct((NW, 128), f32),
        jax.ShapeDtypeStruct((NW, 128), f32),
        jax.ShapeDtypeStruct((1, 128), f32),
    ))(agg0, s0, bn0w, bdn1, bds1)

    agg1 = wide(_segsum(narrow(t1), src2d, dst2d))
    t2, s2, h2sum = _tc(_layer_body, (
        jax.ShapeDtypeStruct((NW, 128), f32),
        jax.ShapeDtypeStruct((NW, 128), f32),
        jax.ShapeDtypeStruct((1, 128), f32),
    ))(agg1, s1, bn1w, bdn2, bds2)

    agg2 = wide(_segsum(narrow(t2), src2d, dst2d))
    out = _tc(_final_body, jax.ShapeDtypeStruct((1, fc_b.shape[0]), f32))(
        agg2, s2, bn2w, xsum, h1sum, h2sum, wx, w1, w2, w3, fcb)
    return out


# speedup vs baseline: 1.0391x; 1.0391x over previous
import functools

import jax
import jax.numpy as jnp
from jax import lax
from jax.experimental import pallas as pl
from jax.experimental.pallas import tpu as pltpu
from jax.experimental.pallas import tpu_sc as plsc

N = 10000
E = 320000
D = 128
H = 16
NPAD = 10112
EPC = 128
CPT = 80
CF = 96
CS = 64
NTILES = 32
EPT = EPC * CPT
EPAD = EPT * NTILES
RPT = NPAD // 16


NB = 4


def _segsum_body(t_hbm, src_hbm, dst_hbm, out_hbm,
                 src_v, dst_v, rows_v, zbuf, acc_sh, t_sh,
                 gsem, gsem2, ssem, ssem2):
    c = lax.axis_index("c")
    s = lax.axis_index("s")

    pltpu.sync_copy(t_hbm.at[pl.ds(s * RPT, RPT)], t_sh.at[pl.ds(s * RPT, RPT)])

    def zrow(j, carry):
        zbuf[j, :] = jnp.zeros((16,), jnp.float32)
        return carry
    lax.fori_loop(0, RPT, zrow, 0)
    pltpu.sync_copy(zbuf, acc_sh.at[pl.ds(s * RPT, RPT)])

    @pl.when(c == 0)
    def _():
        pltpu.sync_copy(src_hbm.at[pl.ds(s * CF, CF)], src_v)
        pltpu.sync_copy(dst_hbm.at[pl.ds(s * CF, CF)], dst_v)

    @pl.when(c == 1)
    def _():
        pltpu.sync_copy(src_hbm.at[pl.ds(16 * CF + s * CS, CS)],
                        src_v.at[pl.ds(0, CS)])
        pltpu.sync_copy(dst_hbm.at[pl.ds(16 * CF + s * CS, CS)],
                        dst_v.at[pl.ds(0, CS)])
    plsc.subcore_barrier()

    ngrp2 = jnp.where(c == 0, CF // NB // 2, CS // NB // 2)

    def fire_gathers(grp, st, sem):
        for b in range(NB):
            pltpu.async_copy(t_sh.at[src_v.at[grp * NB + b]],
                             rows_v.at[st, b], sem)

    def drain(st, sem, kind):
        for b in range(NB):
            if kind == "g":
                pltpu.make_async_copy(t_sh.at[src_v.at[b]],
                                      rows_v.at[st, b], sem).wait()
            else:
                pltpu.make_async_copy(rows_v.at[st, b],
                                      acc_sh.at[dst_v.at[b]], sem).wait()

    def fire_scatters(grp, st, sem):
        for b in range(NB):
            pltpu.async_copy(rows_v.at[st, b],
                             acc_sh.at[dst_v.at[grp * NB + b]],
                             sem, add=True)

    fire_gathers(0, 0, gsem)

    def chunkgrp2(i, carry):
        g0 = 2 * i
        drain(0, gsem, "g")
        @pl.when(i >= 1)
        def _():
            drain(1, ssem2, "s")
        fire_gathers(g0 + 1, 1, gsem2)
        fire_scatters(g0, 0, ssem)
        drain(1, gsem2, "g")
        drain(0, ssem, "s")
        @pl.when(g0 + 2 < 2 * ngrp2)
        def _():
            fire_gathers(g0 + 2, 0, gsem)
        fire_scatters(g0 + 1, 1, ssem2)
        return carry
    lax.fori_loop(0, ngrp2, chunkgrp2, 0)
    drain(1, ssem2, "s")
    plsc.subcore_barrier()

    pltpu.sync_copy(acc_sh.at[pl.ds(s * RPT, RPT)], zbuf)
    pltpu.sync_copy(zbuf, out_hbm.at[c, pl.ds(s * RPT, RPT)])


_segsum = functools.partial(
    pl.kernel,
    _segsum_body,
    out_type=jax.ShapeDtypeStruct((2, NPAD, H), jnp.float32),
    mesh=plsc.VectorSubcoreMesh(core_axis_name="c", subcore_axis_name="s"),
    compiler_params=pltpu.CompilerParams(use_tc_tiling_on_sc=False),
    scratch_types=[
        pltpu.VMEM((CF, EPC), jnp.int32),
        pltpu.VMEM((CF, EPC), jnp.int32),
        pltpu.VMEM((2, NB, EPC, H), jnp.float32),
        pltpu.VMEM((RPT, H), jnp.float32),
        pltpu.VMEM_SHARED((NPAD, H), jnp.float32),
        pltpu.VMEM_SHARED((NPAD, H), jnp.float32),
        pltpu.SemaphoreType.DMA,
        pltpu.SemaphoreType.DMA,
        pltpu.SemaphoreType.DMA,
        pltpu.SemaphoreType.DMA,
    ],
)()


NW = NPAD // 8
NWVAL = N // 8


def _mm0_body(x8_ref, bdn_ref, bds_ref, t_ref, s_ref, xsum_ref):
    xx = x8_ref[...]
    t_ref[...] = jnp.dot(xx, bdn_ref[...], preferred_element_type=jnp.float32)
    s_ref[...] = jnp.dot(xx, bds_ref[...], preferred_element_type=jnp.float32)
    xsum_ref[...] = jnp.sum(xx, axis=0, keepdims=True)


def _layer_body(agg_ref, sprev_ref, bn_ref, bdn_ref, bds_ref,
                t_ref, s_ref, hsum_ref):
    h = jax.nn.relu(agg_ref[0] + agg_ref[1] + bn_ref[...] + sprev_ref[...])
    row = lax.broadcasted_iota(jnp.int32, (NW, 128), 0)
    hm = jnp.where(row < NWVAL, h, 0.0)
    hsum_ref[...] = jnp.sum(hm, axis=0, keepdims=True)
    t_ref[...] = jnp.dot(h, bdn_ref[...], preferred_element_type=jnp.float32)
    s_ref[...] = jnp.dot(h, bds_ref[...], preferred_element_type=jnp.float32)


def _final_body(agg_ref, sprev_ref, bn_ref, xsum_ref, h1s_ref, h2s_ref,
                wx_ref, w1_ref, w2_ref, w3_ref, fcb_ref, out_ref):
    h = jax.nn.relu(agg_ref[0] + agg_ref[1] + bn_ref[...] + sprev_ref[...])
    row = lax.broadcasted_iota(jnp.int32, (NW, 128), 0)
    hm = jnp.where(row < NWVAL, h, 0.0)
    h3sum = jnp.sum(hm, axis=0, keepdims=True)
    logits = (
        jnp.dot(xsum_ref[...], wx_ref[...], preferred_element_type=jnp.float32)
        + jnp.dot(h1s_ref[...], w1_ref[...], preferred_element_type=jnp.float32)
        + jnp.dot(h2s_ref[...], w2_ref[...], preferred_element_type=jnp.float32)
        + jnp.dot(h3sum, w3_ref[...], preferred_element_type=jnp.float32)
        + fcb_ref[...]
    )
    m = jnp.max(logits, axis=1, keepdims=True)
    lse = jnp.log(jnp.sum(jnp.exp(logits - m), axis=1, keepdims=True)) + m
    out_ref[...] = logits - lse


def _tc(body, out_shapes):
    return pl.pallas_call(body, out_shape=out_shapes)


def kernel(x, edge_index, Wn0, bn0, Ws0, Wn1, bn1, Ws1, Wn2, bn2, Ws2,
           fc_W, fc_b):
    f32 = jnp.float32
    x8 = jnp.pad(x, ((0, NPAD - N), (0, 0))).reshape(NW, 8 * D)
    npadchunk = (EPAD - E) // EPC
    pi2 = lax.broadcasted_iota(jnp.int32, (npadchunk, EPC), 1) % 16
    src2d = jnp.concatenate([edge_index[0].reshape(E // EPC, EPC), pi2], axis=0)
    dst2d = jnp.concatenate([edge_index[1].reshape(E // EPC, EPC), N + pi2], axis=0)
    eye8 = jnp.eye(8, dtype=f32)
    bdn0, bds0 = jnp.kron(eye8, Wn0), jnp.kron(eye8, Ws0)
    bdn1, bds1 = jnp.kron(eye8, Wn1), jnp.kron(eye8, Ws1)
    bdn2, bds2 = jnp.kron(eye8, Wn2), jnp.kron(eye8, Ws2)
    bn0w = jnp.tile(bn0, 8).reshape(1, 128)
    bn1w = jnp.tile(bn1, 8).reshape(1, 128)
    bn2w = jnp.tile(bn2, 8).reshape(1, 128)
    fcb = fc_b.reshape(1, -1)
    wx = jnp.tile(fc_W[:D], (8, 1))
    w1 = jnp.tile(fc_W[D:D + H], (8, 1))
    w2 = jnp.tile(fc_W[D + H:D + 2 * H], (8, 1))
    w3 = jnp.tile(fc_W[D + 2 * H:], (8, 1))

    def narrow(a):
        return a.reshape(NPAD, H)

    def wide(a):
        return a.reshape(2, NW, 128)

    t0, s0, xsum = _tc(_mm0_body, (
        jax.ShapeDtypeStruct((NW, 128), f32),
        jax.ShapeDtypeStruct((NW, 128), f32),
        jax.ShapeDtypeStruct((1, 8 * D), f32),
    ))(x8, bdn0, bds0)

    agg0 = wide(_segsum(narrow(t0), src2d, dst2d))
    t1, s1, h1sum = _tc(_layer_body, (
        jax.ShapeDtypeStruct((NW, 128), f32),
        jax.ShapeDtypeStruct((NW, 128), f32),
        jax.ShapeDtypeStruct((1, 128), f32),
    ))(agg0, s0, bn0w, bdn1, bds1)

    agg1 = wide(_segsum(narrow(t1), src2d, dst2d))
    t2, s2, h2sum = _tc(_layer_body, (
        jax.ShapeDtypeStruct((NW, 128), f32),
        jax.ShapeDtypeStruct((NW, 128), f32),
        jax.ShapeDtypeStruct((1, 128), f32),
    ))(agg1, s1, bn1w, bdn2, bds2)

    agg2 = wide(_segsum(narrow(t2), src2d, dst2d))
    out = _tc(_final_body, jax.ShapeDtypeStruct((1, fc_b.shape[0]), f32))(
        agg2, s2, bn2w, xsum, h1sum, h2sum, wx, w1, w2, w3, fcb)
    return out
